# baseline (speedup 1.0000x reference)
import numpy as np
import jax
import jax.numpy as jnp
from jax import lax
from jax.experimental import pallas as pl
from jax.experimental.pallas import tpu as pltpu
from jax.experimental.pallas import tpu_sc as plsc

N_MEM_C = 1000000
B_C = 4096
TWOB_C = 8192
ALPHA_C = np.float32(0.5)

_MOM = np.float32(N_MEM_C) / (np.float32(N_MEM_C) + np.float32(B_C))
_ONE_MINUS_MOM = np.float32(1.0) - _MOM


SC_ROWS = 512
TC_ROWS = B_C - SC_ROWS


RED_ROWS = 256
RED_G = TC_ROWS // RED_ROWS
RED_OFF = SC_ROWS // RED_ROWS


def _reduce_body(qii_ref, qij_ref, qji_ref, sums_ref, acc_ref):
    step = pl.program_id(0)

    @pl.when(step == 0)
    def _init():
        acc_ref[0, 0] = jnp.float32(0.0)

    acc_ref[0, 0] += jnp.sum(qij_ref[...]) + jnp.sum(qji_ref[...])

    @pl.when(step == RED_G - 1)
    def _finish():
        s = acc_ref[0, 0]
        sii = jnp.sum(qii_ref[...])
        for j in range(16):
            sums_ref[j] = s
            sums_ref[16 + j] = sii


def _reduce_tc(qii2d, qij, qji):
    return pl.pallas_call(
        _reduce_body,
        grid=(RED_G,),
        in_specs=[
            pl.BlockSpec((32, 128), lambda i: (0, 0)),
            pl.BlockSpec((RED_ROWS, TWOB_C), lambda i: (i + RED_OFF, 0)),
            pl.BlockSpec((RED_ROWS, TWOB_C), lambda i: (i + RED_OFF, 0)),
        ],
        out_specs=pl.BlockSpec(memory_space=pltpu.SMEM),
        out_shape=jax.ShapeDtypeStruct((32,), jnp.float32),
        scratch_shapes=[pltpu.SMEM((1, 1), jnp.float32)],
    )(qii2d, qij, qji)


NC = 2
NS = 16
NW = NC * NS
L = 16
PERW = B_C // NW
CHUNK = 31264
TAIL = N_MEM_C - (NW - 1) * CHUNK
CH = 32768
REDW = SC_ROWS * TWOB_C // NS
NCH = REDW // CH

_SC_PARAMS = pltpu.CompilerParams(needs_layout_passes=False)


def _accum_chunk(buf_v, accs):
    def vec4(i, a):
        a0, a1, a2, a3 = a
        o = pl.multiple_of(i * (4 * L), L)
        a0 = a0 + buf_v[pl.ds(o, L)]
        a1 = a1 + buf_v[pl.ds(o + L, L)]
        a2 = a2 + buf_v[pl.ds(o + 2 * L, L)]
        a3 = a3 + buf_v[pl.ds(o + 3 * L, L)]
        return (a0, a1, a2, a3)

    return lax.fori_loop(0, CH // (4 * L), vec4, accs, unroll=2)


def _front_body(idx_hbm, sinv_hbm, qij_hbm, qji_hbm, out_hbm, pg_hbm, part_hbm,
                buf_v, idx_v, pg_v, st_v, sem):
    cid = lax.axis_index("c")
    sid = lax.axis_index("s")
    wid = sid * NC + cid

    base = pl.multiple_of(wid * PERW, 8)
    pltpu.sync_copy(idx_hbm.at[pl.ds(base, PERW)], idx_v)
    pltpu.async_copy(sinv_hbm.at[idx_v], pg_v, sem).wait()
    for j in range(PERW // L):
        pg_v[pl.ds(j * L, L)] = pg_v[pl.ds(j * L, L)] * jnp.float32(_ONE_MINUS_MOM)
    pltpu.sync_copy(pg_v, pg_hbm.at[pl.ds(base, PERW)])

    lane = jnp.where(wid < NS, wid, wid - NS)
    rstart = pl.multiple_of(lane * REDW, 8)
    accs = (jnp.zeros((L,), jnp.float32),) * 4
    for ch in range(NCH):
        off = pl.multiple_of(rstart + ch * CH, 8)

        @pl.when(wid < NS)
        def _ij():
            pltpu.sync_copy(qij_hbm.at[pl.ds(off, CH)], buf_v)

        @pl.when(wid >= NS)
        def _ji():
            pltpu.sync_copy(qji_hbm.at[pl.ds(off, CH)], buf_v)

        accs = _accum_chunk(buf_v, accs)
    tot = (accs[0] + accs[1]) + (accs[2] + accs[3])
    st_v[...] = tot
    pltpu.sync_copy(st_v, part_hbm.at[pl.ds(wid * L, L)])

    cbase = pl.multiple_of(wid * CHUNK, 8)

    @pl.when(wid < NW - 1)
    def _copy_main():
        pltpu.sync_copy(sinv_hbm.at[pl.ds(cbase, CHUNK)], buf_v.at[pl.ds(0, CHUNK)])
        pltpu.sync_copy(buf_v.at[pl.ds(0, CHUNK)], out_hbm.at[pl.ds(cbase, CHUNK)])

    @pl.when(wid == NW - 1)
    def _copy_tail():
        pltpu.sync_copy(sinv_hbm.at[pl.ds(cbase, TAIL)], buf_v.at[pl.ds(0, TAIL)])
        pltpu.sync_copy(buf_v.at[pl.ds(0, TAIL)], out_hbm.at[pl.ds(cbase, TAIL)])


def _sc_front(idx32, s_inv, qij1d, qji1d, out_ref):
    mesh = plsc.VectorSubcoreMesh(core_axis_name="c", subcore_axis_name="s")
    f = pl.kernel(
        _front_body,
        out_type=(
            jax.ShapeDtypeStruct((B_C,), jnp.float32),
            jax.ShapeDtypeStruct((NW * L,), jnp.float32),
        ),
        mesh=mesh,
        scratch_types=[
            pltpu.VMEM((CH,), jnp.float32),
            pltpu.VMEM((PERW,), jnp.int32),
            pltpu.VMEM((PERW,), jnp.float32),
            pltpu.VMEM((L,), jnp.float32),
            pltpu.SemaphoreType.DMA,
        ],
        compiler_params=_SC_PARAMS,
    )
    return f(idx32, s_inv, qij1d, qji1d, out_ref)


def _finish_body(idx_hbm, pg_hbm, sums_hbm, part_hbm, out_hbm,
                 idx_v, pg_v, sums_v, part_v, sem):
    cid = lax.axis_index("c")
    sid = lax.axis_index("s")
    wid = sid * NC + cid
    base = pl.multiple_of(wid * PERW, 8)
    pltpu.sync_copy(idx_hbm.at[pl.ds(base, PERW)], idx_v)
    pltpu.sync_copy(pg_hbm.at[pl.ds(base, PERW)], pg_v)
    pltpu.sync_copy(sums_hbm, sums_v)
    pltpu.sync_copy(part_hbm, part_v)

    p = part_v[pl.ds(0, L)]
    for k in range(1, NW):
        p = p + part_v[pl.ds(k * L, L)]
    s_sc = lax.reduce_sum(p, axes=(0,))
    s_tc = sums_v[pl.ds(0, L)][0]
    sii = sums_v[pl.ds(L, L)][0]
    nf = jnp.float32(N_MEM_C)
    alpha = jnp.float32(ALPHA_C)
    inv4b = jnp.float32(1.0 / (4.0 * B_C))
    invb = jnp.float32(1.0 / B_C)
    s = s_tc + s_sc
    xi = alpha * sii + (jnp.float32(1.0) - alpha) * (s * inv4b)
    c = jnp.float32(_MOM) * nf * (xi * invb)
    cvec = jnp.broadcast_to(c, (L,))

    for j in range(PERW // L):
        pg_v[pl.ds(j * L, L)] = pg_v[pl.ds(j * L, L)] + cvec
    pltpu.async_copy(pg_v, out_hbm.at[idx_v], sem).wait()


def _sc_finish(idx32, pg, sums, part, out_ref):
    mesh = plsc.VectorSubcoreMesh(core_axis_name="c", subcore_axis_name="s")
    f = pl.kernel(
        _finish_body,
        out_type=(),
        mesh=mesh,
        scratch_types=[
            pltpu.VMEM((PERW,), jnp.int32),
            pltpu.VMEM((PERW,), jnp.float32),
            pltpu.VMEM((2 * L,), jnp.float32),
            pltpu.VMEM((NW * L,), jnp.float32),
            pltpu.SemaphoreType.DMA,
        ],
        compiler_params=_SC_PARAMS,
    )
    f(idx32, pg, sums, part, out_ref)


def kernel(qii, qij, qji, feats_idx, s_inv):
    idx32 = feats_idx.astype(jnp.int32)
    qij1d = qij.reshape(B_C * TWOB_C)
    qji1d = qji.reshape(B_C * TWOB_C)
    out_ref = jax.empty_ref(jax.ShapeDtypeStruct((N_MEM_C,), jnp.float32))
    pg, part = _sc_front(idx32, s_inv, qij1d, qji1d, out_ref)
    qii2d = qii.reshape(32, 128)
    sums = _reduce_tc(qii2d, qij, qji)
    _sc_finish(idx32, pg, sums, part, out_ref)
    return out_ref[...]


# speedup vs baseline: 2.7190x; 2.7190x over previous
import numpy as np
import jax
import jax.numpy as jnp
from jax import lax
from jax.experimental import pallas as pl
from jax.experimental.pallas import tpu as pltpu
from jax.experimental.pallas import tpu_sc as plsc

N_MEM_C = 1000000
B_C = 4096
TWOB_C = 8192
ALPHA_C = np.float32(0.5)

_MOM = np.float32(N_MEM_C) / (np.float32(N_MEM_C) + np.float32(B_C))
_ONE_MINUS_MOM = np.float32(1.0) - _MOM


SC_ROWS = 0
TC_ROWS = B_C - SC_ROWS


RED_ROWS = 256
RED_G = TC_ROWS // RED_ROWS
RED_OFF = SC_ROWS // RED_ROWS


def _reduce_body(qii_ref, qij_ref, qji_ref, sums_ref, acc_ref):
    step = pl.program_id(0)

    @pl.when(step == 0)
    def _init():
        acc_ref[0, 0] = jnp.float32(0.0)

    acc_ref[0, 0] += jnp.sum(qij_ref[...]) + jnp.sum(qji_ref[...])

    @pl.when(step == RED_G - 1)
    def _finish():
        s = acc_ref[0, 0]
        sii = jnp.sum(qii_ref[...])
        for j in range(16):
            sums_ref[j] = s
            sums_ref[16 + j] = sii


def _reduce_tc(qii2d, qij, qji):
    return pl.pallas_call(
        _reduce_body,
        grid=(RED_G,),
        in_specs=[
            pl.BlockSpec((32, 128), lambda i: (0, 0)),
            pl.BlockSpec((RED_ROWS, TWOB_C), lambda i: (i + RED_OFF, 0)),
            pl.BlockSpec((RED_ROWS, TWOB_C), lambda i: (i + RED_OFF, 0)),
        ],
        out_specs=pl.BlockSpec(memory_space=pltpu.SMEM),
        out_shape=jax.ShapeDtypeStruct((32,), jnp.float32),
        scratch_shapes=[pltpu.SMEM((1, 1), jnp.float32)],
    )(qii2d, qij, qji)


NC = 2
NS = 16
NW = NC * NS
L = 16
PERW = B_C // NW
CHUNK = 31264
TAIL = N_MEM_C - (NW - 1) * CHUNK
CH = 32768
REDW = SC_ROWS * TWOB_C // NS
NCH = REDW // CH

_SC_PARAMS = pltpu.CompilerParams(needs_layout_passes=False)


def _accum_chunk(buf_v, accs):
    def vec4(i, a):
        a0, a1, a2, a3 = a
        o = pl.multiple_of(i * (4 * L), L)
        a0 = a0 + buf_v[pl.ds(o, L)]
        a1 = a1 + buf_v[pl.ds(o + L, L)]
        a2 = a2 + buf_v[pl.ds(o + 2 * L, L)]
        a3 = a3 + buf_v[pl.ds(o + 3 * L, L)]
        return (a0, a1, a2, a3)

    return lax.fori_loop(0, CH // (4 * L), vec4, accs, unroll=2)


def _front_body(idx_hbm, sinv_hbm, out_hbm, pg_hbm,
                buf_v, idx_v, pg_v, sem):
    cid = lax.axis_index("c")
    sid = lax.axis_index("s")
    wid = sid * NC + cid

    base = pl.multiple_of(wid * PERW, 8)
    pltpu.sync_copy(idx_hbm.at[pl.ds(base, PERW)], idx_v)
    pltpu.async_copy(sinv_hbm.at[idx_v], pg_v, sem).wait()
    for j in range(PERW // L):
        pg_v[pl.ds(j * L, L)] = pg_v[pl.ds(j * L, L)] * jnp.float32(_ONE_MINUS_MOM)
    pltpu.sync_copy(pg_v, pg_hbm.at[pl.ds(base, PERW)])

    cbase = pl.multiple_of(wid * CHUNK, 8)

    @pl.when(wid < NW - 1)
    def _copy_main():
        pltpu.sync_copy(sinv_hbm.at[pl.ds(cbase, CHUNK)], buf_v.at[pl.ds(0, CHUNK)])
        pltpu.sync_copy(buf_v.at[pl.ds(0, CHUNK)], out_hbm.at[pl.ds(cbase, CHUNK)])

    @pl.when(wid == NW - 1)
    def _copy_tail():
        pltpu.sync_copy(sinv_hbm.at[pl.ds(cbase, TAIL)], buf_v.at[pl.ds(0, TAIL)])
        pltpu.sync_copy(buf_v.at[pl.ds(0, TAIL)], out_hbm.at[pl.ds(cbase, TAIL)])


def _sc_front(idx32, s_inv, out_ref):
    mesh = plsc.VectorSubcoreMesh(core_axis_name="c", subcore_axis_name="s")
    f = pl.kernel(
        _front_body,
        out_type=jax.ShapeDtypeStruct((B_C,), jnp.float32),
        mesh=mesh,
        scratch_types=[
            pltpu.VMEM((CH,), jnp.float32),
            pltpu.VMEM((PERW,), jnp.int32),
            pltpu.VMEM((PERW,), jnp.float32),
            pltpu.SemaphoreType.DMA,
        ],
        compiler_params=_SC_PARAMS,
    )
    return f(idx32, s_inv, out_ref)


def _finish_body(idx_hbm, pg_hbm, sums_hbm, out_hbm,
                 idx_v, pg_v, sums_v, sem):
    cid = lax.axis_index("c")
    sid = lax.axis_index("s")
    wid = sid * NC + cid
    base = pl.multiple_of(wid * PERW, 8)
    pltpu.sync_copy(idx_hbm.at[pl.ds(base, PERW)], idx_v)
    pltpu.sync_copy(pg_hbm.at[pl.ds(base, PERW)], pg_v)
    pltpu.sync_copy(sums_hbm, sums_v)

    s_tc = sums_v[pl.ds(0, L)][0]
    sii = sums_v[pl.ds(L, L)][0]
    nf = jnp.float32(N_MEM_C)
    alpha = jnp.float32(ALPHA_C)
    inv4b = jnp.float32(1.0 / (4.0 * B_C))
    invb = jnp.float32(1.0 / B_C)
    s = s_tc
    xi = alpha * sii + (jnp.float32(1.0) - alpha) * (s * inv4b)
    c = jnp.float32(_MOM) * nf * (xi * invb)
    cvec = jnp.broadcast_to(c, (L,))

    for j in range(PERW // L):
        pg_v[pl.ds(j * L, L)] = pg_v[pl.ds(j * L, L)] + cvec
    pltpu.async_copy(pg_v, out_hbm.at[idx_v], sem).wait()


def _sc_finish(idx32, pg, sums, out_ref):
    mesh = plsc.VectorSubcoreMesh(core_axis_name="c", subcore_axis_name="s")
    f = pl.kernel(
        _finish_body,
        out_type=(),
        mesh=mesh,
        scratch_types=[
            pltpu.VMEM((PERW,), jnp.int32),
            pltpu.VMEM((PERW,), jnp.float32),
            pltpu.VMEM((2 * L,), jnp.float32),
            pltpu.SemaphoreType.DMA,
        ],
        compiler_params=_SC_PARAMS,
    )
    f(idx32, pg, sums, out_ref)


def kernel(qii, qij, qji, feats_idx, s_inv):
    idx32 = feats_idx.astype(jnp.int32)
    out_ref = jax.empty_ref(jax.ShapeDtypeStruct((N_MEM_C,), jnp.float32))
    pg = _sc_front(idx32, s_inv, out_ref)
    qii2d = qii.reshape(32, 128)
    sums = _reduce_tc(qii2d, qij, qji)
    _sc_finish(idx32, pg, sums, out_ref)
    return out_ref[...]


# speedup vs baseline: 2.9000x; 1.0666x over previous
import numpy as np
import jax
import jax.numpy as jnp
from jax import lax
from jax.experimental import pallas as pl
from jax.experimental.pallas import tpu as pltpu
from jax.experimental.pallas import tpu_sc as plsc

N_MEM_C = 1000000
B_C = 4096
TWOB_C = 8192
ALPHA_C = np.float32(0.5)

_MOM = np.float32(N_MEM_C) / (np.float32(N_MEM_C) + np.float32(B_C))
_ONE_MINUS_MOM = np.float32(1.0) - _MOM


SC_ROWS = 512
TC_ROWS = B_C - SC_ROWS


RED_ROWS = 256
RED_G = TC_ROWS // RED_ROWS
RED_OFF = SC_ROWS // RED_ROWS


def _reduce_body(qii_ref, qij_ref, qji_ref, sums_ref, acc_ref):
    step = pl.program_id(0)

    @pl.when(step == 0)
    def _init():
        acc_ref[0, 0] = jnp.float32(0.0)

    acc_ref[0, 0] += jnp.sum(qij_ref[...]) + jnp.sum(qji_ref[...])

    @pl.when(step == RED_G - 1)
    def _finish():
        s = acc_ref[0, 0]
        sii = jnp.sum(qii_ref[...])
        for j in range(16):
            sums_ref[j] = s
            sums_ref[16 + j] = sii


def _reduce_tc(qii2d, qij, qji):
    return pl.pallas_call(
        _reduce_body,
        grid=(RED_G,),
        in_specs=[
            pl.BlockSpec((32, 128), lambda i: (0, 0)),
            pl.BlockSpec((RED_ROWS, TWOB_C), lambda i: (i + RED_OFF, 0)),
            pl.BlockSpec((RED_ROWS, TWOB_C), lambda i: (i + RED_OFF, 0)),
        ],
        out_specs=pl.BlockSpec(memory_space=pltpu.SMEM),
        out_shape=jax.ShapeDtypeStruct((32,), jnp.float32),
        scratch_shapes=[pltpu.SMEM((1, 1), jnp.float32)],
    )(qii2d, qij, qji)


NC = 2
NS = 16
NW = NC * NS
L = 16
PERW = B_C // NW
CHUNK = 31264
TAIL = N_MEM_C - (NW - 1) * CHUNK
ROWS_PW = SC_ROWS // NS
NBAND = ROWS_PW // 8

_SC_PARAMS = pltpu.CompilerParams(needs_layout_passes=False)


def _accum_band(band_v, accs):
    def vec4(i, a):
        a0, a1, a2, a3 = a
        o = pl.multiple_of(i * (4 * L), L)
        r = i // (TWOB_C // (4 * L))
        oc = o % TWOB_C
        a0 = a0 + band_v[r, pl.ds(oc, L)]
        a1 = a1 + band_v[r, pl.ds(oc + L, L)]
        a2 = a2 + band_v[r, pl.ds(oc + 2 * L, L)]
        a3 = a3 + band_v[r, pl.ds(oc + 3 * L, L)]
        return (a0, a1, a2, a3)

    return lax.fori_loop(0, 8 * TWOB_C // (4 * L), vec4, accs, unroll=2)


def _front_body(idx_hbm, sinv_hbm, qij_hbm, qji_hbm, out_hbm, pg_hbm, part_hbm,
                buf_v, band_v, idx_v, pg_v, st_v, sem):
    cid = lax.axis_index("c")
    sid = lax.axis_index("s")
    wid = sid * NC + cid

    base = pl.multiple_of(wid * PERW, 8)
    pltpu.sync_copy(idx_hbm.at[pl.ds(base, PERW)], idx_v)
    pltpu.async_copy(sinv_hbm.at[idx_v], pg_v, sem).wait()
    for j in range(PERW // L):
        pg_v[pl.ds(j * L, L)] = pg_v[pl.ds(j * L, L)] * jnp.float32(_ONE_MINUS_MOM)
    pltpu.sync_copy(pg_v, pg_hbm.at[pl.ds(base, PERW)])

    lane = jnp.where(wid < NS, wid, wid - NS)
    r0 = lane * ROWS_PW
    accs = (jnp.zeros((L,), jnp.float32),) * 4
    for band in range(NBAND):
        rb = r0 + band * 8

        @pl.when(wid < NS)
        def _ij():
            pltpu.sync_copy(qij_hbm.at[pl.ds(rb, 8), :], band_v)

        @pl.when(wid >= NS)
        def _ji():
            pltpu.sync_copy(qji_hbm.at[pl.ds(rb, 8), :], band_v)

        accs = _accum_band(band_v, accs)
    tot = (accs[0] + accs[1]) + (accs[2] + accs[3])
    st_v[...] = tot
    pltpu.sync_copy(st_v, part_hbm.at[pl.ds(wid * L, L)])

    cbase = pl.multiple_of(wid * CHUNK, 8)

    @pl.when(wid < NW - 1)
    def _copy_main():
        pltpu.sync_copy(sinv_hbm.at[pl.ds(cbase, CHUNK)], buf_v.at[pl.ds(0, CHUNK)])
        pltpu.sync_copy(buf_v.at[pl.ds(0, CHUNK)], out_hbm.at[pl.ds(cbase, CHUNK)])

    @pl.when(wid == NW - 1)
    def _copy_tail():
        pltpu.sync_copy(sinv_hbm.at[pl.ds(cbase, TAIL)], buf_v.at[pl.ds(0, TAIL)])
        pltpu.sync_copy(buf_v.at[pl.ds(0, TAIL)], out_hbm.at[pl.ds(cbase, TAIL)])


def _sc_front(idx32, s_inv, qij, qji, out_ref):
    mesh = plsc.VectorSubcoreMesh(core_axis_name="c", subcore_axis_name="s")
    f = pl.kernel(
        _front_body,
        out_type=(
            jax.ShapeDtypeStruct((B_C,), jnp.float32),
            jax.ShapeDtypeStruct((NW * L,), jnp.float32),
        ),
        mesh=mesh,
        scratch_types=[
            pltpu.VMEM((CHUNK,), jnp.float32),
            pltpu.VMEM((8, TWOB_C), jnp.float32),
            pltpu.VMEM((PERW,), jnp.int32),
            pltpu.VMEM((PERW,), jnp.float32),
            pltpu.VMEM((L,), jnp.float32),
            pltpu.SemaphoreType.DMA,
        ],
        compiler_params=_SC_PARAMS,
    )
    return f(idx32, s_inv, qij, qji, out_ref)


def _finish_body(idx_hbm, pg_hbm, sums_hbm, part_hbm, out_hbm,
                 idx_v, pg_v, sums_v, part_v, sem):
    cid = lax.axis_index("c")
    sid = lax.axis_index("s")
    wid = sid * NC + cid
    base = pl.multiple_of(wid * PERW, 8)
    pltpu.sync_copy(idx_hbm.at[pl.ds(base, PERW)], idx_v)
    pltpu.sync_copy(pg_hbm.at[pl.ds(base, PERW)], pg_v)
    pltpu.sync_copy(sums_hbm, sums_v)
    pltpu.sync_copy(part_hbm, part_v)

    p = part_v[pl.ds(0, L)]
    for k in range(1, NW):
        p = p + part_v[pl.ds(k * L, L)]
    s_sc = lax.reduce_sum(p, axes=(0,))
    s_tc = sums_v[pl.ds(0, L)][0]
    sii = sums_v[pl.ds(L, L)][0]
    nf = jnp.float32(N_MEM_C)
    alpha = jnp.float32(ALPHA_C)
    inv4b = jnp.float32(1.0 / (4.0 * B_C))
    invb = jnp.float32(1.0 / B_C)
    s = s_tc + s_sc
    xi = alpha * sii + (jnp.float32(1.0) - alpha) * (s * inv4b)
    c = jnp.float32(_MOM) * nf * (xi * invb)
    cvec = jnp.broadcast_to(c, (L,))

    for j in range(PERW // L):
        pg_v[pl.ds(j * L, L)] = pg_v[pl.ds(j * L, L)] + cvec
    pltpu.async_copy(pg_v, out_hbm.at[idx_v], sem).wait()


def _sc_finish(idx32, pg, sums, part, out_ref):
    mesh = plsc.VectorSubcoreMesh(core_axis_name="c", subcore_axis_name="s")
    f = pl.kernel(
        _finish_body,
        out_type=(),
        mesh=mesh,
        scratch_types=[
            pltpu.VMEM((PERW,), jnp.int32),
            pltpu.VMEM((PERW,), jnp.float32),
            pltpu.VMEM((2 * L,), jnp.float32),
            pltpu.VMEM((NW * L,), jnp.float32),
            pltpu.SemaphoreType.DMA,
        ],
        compiler_params=_SC_PARAMS,
    )
    f(idx32, pg, sums, part, out_ref)


def kernel(qii, qij, qji, feats_idx, s_inv):
    idx32 = feats_idx.astype(jnp.int32)
    out_ref = jax.empty_ref(jax.ShapeDtypeStruct((N_MEM_C,), jnp.float32))
    pg, part = _sc_front(idx32, s_inv, qij, qji, out_ref)
    qii2d = qii.reshape(32, 128)
    sums = _reduce_tc(qii2d, qij, qji)
    _sc_finish(idx32, pg, sums, part, out_ref)
    return out_ref[...]
